# Initial kernel scaffold; baseline (speedup 1.0000x reference)
import jax
import jax.numpy as jnp
from jax.experimental import pallas as pl


def kernel(x_q, x_k):
    raise NotImplementedError("write your pallas kernel here")


# speedup vs baseline: 1.8816x; 1.8816x over previous
import functools
import math

import jax
import jax.numpy as jnp
import numpy as np
from jax import lax
from jax.experimental import pallas as pl
from jax.experimental.pallas import tpu as pltpu

B = 2048
D = 128
N_HARD = 256
S1_HARD = 64
INV_T = 2.0
TAU_PLUS = 0.1
NVAL = 2 * B - 2 + S1_HARD
NG_FLOOR = NVAL * math.e ** (-INV_T)

_IDXS1 = np.asarray(
    jax.random.randint(jax.random.key(42), (2 * B, S1_HARD), 0, N_HARD)
)
_W = np.zeros((2 * B, N_HARD), dtype=np.float32)
np.add.at(_W, (np.arange(2 * B)[:, None], _IDXS1), 1.0)

MASK_VAL = -3.0


def _norm_body(xq_ref, xk_ref, out_ref, pos_ref):
    xq = xq_ref[...]
    xk = xk_ref[...]
    n1 = xq / jnp.maximum(jnp.sqrt(jnp.sum(xq * xq, 1, keepdims=True)), 1e-12)
    n2 = xk / jnp.maximum(jnp.sqrt(jnp.sum(xk * xk, 1, keepdims=True)), 1e-12)
    out_ref[:B, :] = n1
    out_ref[B:, :] = n2
    pos_ref[...] = jnp.exp(jnp.sum(n1 * n2, 1) * INV_T)


def _normalize(x_q, x_k):
    return pl.pallas_call(
        _norm_body,
        out_shape=(
            jax.ShapeDtypeStruct((2 * B, D), jnp.float32),
            jax.ShapeDtypeStruct((B,), jnp.float32),
        ),
    )(x_q, x_k)


_RB = 512
_CB = 2048
_NJ = 2 * B // _CB


def _sim_body(a_ref, bt_ref, sim_ref, ns_ref, nss_ref):
    i = pl.program_id(0)
    j = pl.program_id(1)
    a = a_ref[...]
    bt = bt_ref[...]
    sim = lax.dot_general(
        a, bt, (((1,), (1,)), ((), ())), preferred_element_type=jnp.float32
    )
    r = i * _RB + lax.broadcasted_iota(jnp.int32, (_RB, _CB), 0)
    c = j * _CB + lax.broadcasted_iota(jnp.int32, (_RB, _CB), 1)
    m = (c == r) | (c == r - B) | (c == r + B)
    e = jnp.exp(sim * INV_T)
    e = jnp.where(m, 0.0, e)
    sim_ref[...] = jnp.where(m, MASK_VAL, sim)

    @pl.when(j == 0)
    def _():
        ns_ref[...] = jnp.zeros_like(ns_ref)
        nss_ref[...] = jnp.zeros_like(nss_ref)

    ns_ref[...] += jnp.sum(e, 1)
    nss_ref[...] += jnp.sum(e * e, 1)


def _sim_stats(out):
    return pl.pallas_call(
        _sim_body,
        grid=(2 * B // _RB, _NJ),
        in_specs=[
            pl.BlockSpec((_RB, D), lambda i, j: (i, 0)),
            pl.BlockSpec((_CB, D), lambda i, j: (j, 0)),
        ],
        out_specs=(
            pl.BlockSpec((_RB, _CB), lambda i, j: (i, j)),
            pl.BlockSpec((_RB,), lambda i, j: (i,)),
            pl.BlockSpec((_RB,), lambda i, j: (i,)),
        ),
        out_shape=(
            jax.ShapeDtypeStruct((2 * B, 2 * B), jnp.float32),
            jax.ShapeDtypeStruct((2 * B,), jnp.float32),
            jax.ShapeDtypeStruct((2 * B,), jnp.float32),
        ),
        compiler_params=pltpu.CompilerParams(
            dimension_semantics=("parallel", "arbitrary")
        ),
    )(out, out)


def _loss_body(pos_ref, ns_ref, nss_ref, hs_ref, hss_ref, out_ref):
    pos = jnp.concatenate([pos_ref[...], pos_ref[...]])
    s1 = ns_ref[...] + hs_ref[...]
    s2 = nss_ref[...] + hss_ref[...]
    reweight = NVAL * s2 / s1
    ng = (-TAU_PLUS * NVAL * pos + reweight) / (1.0 - TAU_PLUS)
    ng = jnp.maximum(ng, NG_FLOOR)
    out_ref[...] = jnp.mean(jnp.log((pos + ng) / pos)).reshape(1, 1)


def _loss(pos, ns, nss, hs, hss):
    return pl.pallas_call(
        _loss_body,
        out_shape=jax.ShapeDtypeStruct((1, 1), jnp.float32),
    )(pos, ns, nss, hs, hss)


def _hard_sums(sim_masked, w):
    vals = jnp.exp(lax.top_k(sim_masked, N_HARD)[0] * INV_T)
    return jnp.sum(w * vals, 1), jnp.sum(w * vals * vals, 1)


@jax.jit
def kernel(x_q, x_k):
    out, pos = _normalize(x_q, x_k)
    sim_masked, ns, nss = _sim_stats(out)
    w = jnp.asarray(_W)
    hs, hss = _hard_sums(sim_masked, w)
    return _loss(pos, ns, nss, hs, hss)[0, 0]
